# baseline (speedup 1.0000x reference)
import functools

import jax
import jax.numpy as jnp
from jax import lax
from jax.experimental import pallas as pl
from jax.experimental.pallas import tpu as pltpu
from jax.experimental.pallas import tpu_sc as plsc

N_EDGES = 3200000
E1 = N_EDGES + 1
BATCH = 16384
PATH_LEN = 32

NC, NS = 2, 16
NW = NC * NS
NTOT = BATCH * PATH_LEN
PW = NTOT // NW
BPW = BATCH // NW
LANES = 16

_LOG1P_C = (3.511021357038846e-06, 0.9997923620654405, -0.49697743071892625,
            0.3145891739884515, -0.1887808235475876, 0.08172564528980446,
            -0.017207799230048133)


def _softplus(x):
    t = jnp.exp(-jnp.abs(x))
    p = jnp.full_like(t, _LOG1P_C[6])
    for c in (_LOG1P_C[5], _LOG1P_C[4], _LOG1P_C[3], _LOG1P_C[2],
              _LOG1P_C[1], _LOG1P_C[0]):
        p = p * t + c
    return jnp.maximum(x, 0.0) + p


def _sc_reorder_body(tp_hbm, reorder_hbm, pidx_hbm, idx_v, pidx_v, sem):
    wid = lax.axis_index("s") * NC + lax.axis_index("c")
    base = wid * PW
    pltpu.sync_copy(tp_hbm.at[pl.ds(base, PW)], idx_v)
    pltpu.async_copy(reorder_hbm.at[idx_v], pidx_v, sem).wait()
    pltpu.sync_copy(pidx_v, pidx_hbm.at[pl.ds(base, PW)])


_sc_reorder = functools.partial(
    pl.kernel,
    out_type=jax.ShapeDtypeStruct((NTOT,), jnp.int32),
    mesh=plsc.VectorSubcoreMesh(core_axis_name="c", subcore_axis_name="s"),
    scratch_types=[
        pltpu.VMEM((PW,), jnp.int32),
        pltpu.VMEM((PW,), jnp.int32),
        pltpu.SemaphoreType.DMA,
    ],
    compiler_params=pltpu.CompilerParams(needs_layout_passes=False),
)(_sc_reorder_body)


def _sc_main_body(tp_hbm, pidx_hbm, pk_hbm, fi_hbm, ti_hbm, dflt_hbm,
                  dist_hbm, logp_hbm,
                  idx_v, pidx_v, pk_v, fi_v, ti_v, dflt_v, dist_v, logp_v,
                  sem):
    wid = lax.axis_index("s") * NC + lax.axis_index("c")
    base = wid * PW
    pltpu.sync_copy(tp_hbm.at[pl.ds(base, PW)], idx_v)
    pltpu.sync_copy(pidx_hbm.at[pl.ds(base, PW)], pidx_v)
    pltpu.sync_copy(fi_hbm.at[pl.ds(wid * BPW, BPW)], fi_v)
    pltpu.sync_copy(ti_hbm.at[pl.ds(wid * BPW, BPW)], ti_v)
    pltpu.sync_copy(dflt_hbm, dflt_v)
    pltpu.async_copy(pk_hbm.at[pidx_v], pk_v, sem).wait()

    lane = lax.iota(jnp.int32, LANES)
    zero_f = jnp.zeros((LANES,), jnp.float32)
    hi_mask = jnp.full((LANES,), -65536, jnp.int32)

    def grp(g, _):
        bid = (g * LANES + lane) * PATH_LEN

        def inner(l, carry):
            aw, aa = carry
            ids = bid + l
            m = plsc.load_gather(idx_v, [ids]) != 0
            pk = plsc.load_gather(pk_v, [ids])
            w = plsc.bitcast(pk & hi_mask, jnp.float32)
            a = plsc.bitcast(pk << 16, jnp.float32)
            aw = aw + jnp.where(m, _softplus(w), zero_f)
            aa = aa + jnp.where(m, _softplus(-a), zero_f)
            return aw, aa

        acc_w, acc_a = lax.fori_loop(0, PATH_LEN, inner, (zero_f, zero_f))
        tp0 = plsc.load_gather(idx_v, [bid])
        sl = pl.ds(g * LANES, LANES)
        nf = (tp0 == 0) & (fi_v[sl] != ti_v[sl])
        dist_v[sl] = jnp.where(nf, dflt_v[...], acc_w)
        logp_v[sl] = -acc_a
        return 0

    lax.fori_loop(0, BPW // LANES, grp, 0)
    pltpu.sync_copy(dist_v, dist_hbm.at[pl.ds(wid * BPW, BPW)])
    pltpu.sync_copy(logp_v, logp_hbm.at[pl.ds(wid * BPW, BPW)])


_sc_main = functools.partial(
    pl.kernel,
    out_type=(jax.ShapeDtypeStruct((BATCH,), jnp.float32),
              jax.ShapeDtypeStruct((BATCH,), jnp.float32)),
    mesh=plsc.VectorSubcoreMesh(core_axis_name="c", subcore_axis_name="s"),
    scratch_types=[
        pltpu.VMEM((PW,), jnp.int32),
        pltpu.VMEM((PW,), jnp.int32),
        pltpu.VMEM((PW,), jnp.int32),
        pltpu.VMEM((BPW,), jnp.int32),
        pltpu.VMEM((BPW,), jnp.int32),
        pltpu.VMEM((LANES,), jnp.float32),
        pltpu.VMEM((BPW,), jnp.float32),
        pltpu.VMEM((BPW,), jnp.float32),
        pltpu.SemaphoreType.DMA,
    ],
    compiler_params=pltpu.CompilerParams(needs_layout_passes=False),
)(_sc_main_body)


def _bf16_round(u):
    return (u + jnp.uint32(0x7FFF) + ((u >> 16) & jnp.uint32(1))) \
        & jnp.uint32(0xFFFF0000)


def kernel(edge_adjacency_logits, edge_weight_logits, default_distance,
           reorder, target_paths, from_ix, to_ix):
    tp_flat = target_paths.astype(jnp.int32).reshape(NTOT)
    pidx = _sc_reorder(tp_flat, reorder.astype(jnp.int32))
    ar = jnp.arange(E1, dtype=jnp.int32)
    z = jnp.zeros((E1,), jnp.int32)
    wu = lax.bitcast_convert_type(edge_weight_logits[ar, z], jnp.uint32)
    au = lax.bitcast_convert_type(edge_adjacency_logits[ar, z], jnp.uint32)
    packed = lax.bitcast_convert_type(
        _bf16_round(wu) | (_bf16_round(au) >> 16), jnp.int32)
    dflt16 = jnp.broadcast_to(default_distance.reshape(1), (LANES,))
    return _sc_main(tp_flat, pidx, packed,
                    from_ix.astype(jnp.int32),
                    to_ix.astype(jnp.int32),
                    dflt16)


# speedup vs baseline: 1163.8954x; 1163.8954x over previous
import functools

import jax
import jax.numpy as jnp
from jax import lax
from jax.experimental import pallas as pl
from jax.experimental.pallas import tpu as pltpu
from jax.experimental.pallas import tpu_sc as plsc

N_EDGES = 3200000
E1 = N_EDGES + 1
BATCH = 16384
PATH_LEN = 32

NC, NS = 2, 16
NW = NC * NS
NTOT = BATCH * PATH_LEN
PW = NTOT // NW
BPW = BATCH // NW
LANES = 16

_LOG1P_C = (3.511021357038846e-06, 0.9997923620654405, -0.49697743071892625,
            0.3145891739884515, -0.1887808235475876, 0.08172564528980446,
            -0.017207799230048133)


def _softplus(x):
    t = jnp.exp(-jnp.abs(x))
    p = jnp.full_like(t, _LOG1P_C[6])
    for c in (_LOG1P_C[5], _LOG1P_C[4], _LOG1P_C[3], _LOG1P_C[2],
              _LOG1P_C[1], _LOG1P_C[0]):
        p = p * t + c
    return jnp.maximum(x, 0.0) + p


def _sc_reorder_body(tp_hbm, reorder_hbm, pidx_hbm, idx_v, pidx_v, sem):
    wid = lax.axis_index("s") * NC + lax.axis_index("c")
    base = wid * PW
    pltpu.sync_copy(tp_hbm.at[pl.ds(base, PW)], idx_v)
    pltpu.async_copy(reorder_hbm.at[idx_v], pidx_v, sem).wait()
    pltpu.sync_copy(pidx_v, pidx_hbm.at[pl.ds(base, PW)])


_sc_reorder = functools.partial(
    pl.kernel,
    out_type=jax.ShapeDtypeStruct((NTOT,), jnp.int32),
    mesh=plsc.VectorSubcoreMesh(core_axis_name="c", subcore_axis_name="s"),
    scratch_types=[
        pltpu.VMEM((PW,), jnp.int32),
        pltpu.VMEM((PW,), jnp.int32),
        pltpu.SemaphoreType.DMA,
    ],
    compiler_params=pltpu.CompilerParams(needs_layout_passes=False),
)(_sc_reorder_body)


CH = 4096
NCH = PW // CH
ROWS_CH = CH // PATH_LEN


def _sc_main_body(tp_hbm, pidx_hbm, pk_hbm, fi_hbm, ti_hbm, dflt_hbm,
                  dist_hbm, logp_hbm,
                  idx_v, pidx_v, pk0_v, pk1_v, fi_v, ti_v, dflt_v,
                  dist_v, logp_v, sem0, sem1):
    wid = lax.axis_index("s") * NC + lax.axis_index("c")
    base = wid * PW
    pltpu.sync_copy(tp_hbm.at[pl.ds(base, PW)], idx_v)
    pltpu.sync_copy(pidx_hbm.at[pl.ds(base, PW)], pidx_v)
    pltpu.sync_copy(fi_hbm.at[pl.ds(wid * BPW, BPW)], fi_v)
    pltpu.sync_copy(ti_hbm.at[pl.ds(wid * BPW, BPW)], ti_v)
    pltpu.sync_copy(dflt_hbm, dflt_v)

    sems = (sem0, sem1)
    bufs = (pk0_v, pk1_v)
    lane = lax.iota(jnp.int32, LANES)
    zero_f = jnp.zeros((LANES,), jnp.float32)
    hi_mask = jnp.full((LANES,), -65536, jnp.int32)

    def gather_chunk(q, buf):
        pidx_q = pidx_v.at[pl.ds(q * CH, CH)]
        return pltpu.async_copy(pk_hbm.at[pidx_q], bufs[buf], sems[buf])

    def compute_chunk(q, buf):
        def grp(g, _):
            bid = (g * LANES + lane) * PATH_LEN

            def inner(l, carry):
                aw, aa = carry
                ids = bid + l
                m = plsc.load_gather(idx_v, [q * CH + ids]) != 0
                pk = plsc.load_gather(bufs[buf], [ids])
                w = plsc.bitcast(pk & hi_mask, jnp.float32)
                a = plsc.bitcast(pk << 16, jnp.float32)
                aw = aw + jnp.where(m, _softplus(w), zero_f)
                aa = aa + jnp.where(m, _softplus(-a), zero_f)
                return aw, aa

            acc_w, acc_a = lax.fori_loop(0, PATH_LEN, inner, (zero_f, zero_f))
            tp0 = plsc.load_gather(idx_v, [q * CH + bid])
            sl = pl.ds(q * ROWS_CH + g * LANES, LANES)
            nf = (tp0 == 0) & (fi_v[sl] != ti_v[sl])
            dist_v[sl] = jnp.where(nf, dflt_v[...], acc_w)
            logp_v[sl] = -acc_a
            return 0

        lax.fori_loop(0, ROWS_CH // LANES, grp, 0)

    copy = gather_chunk(0, 0)
    for q in range(NCH):
        nxt = gather_chunk(q + 1, (q + 1) % 2) if q + 1 < NCH else None
        copy.wait()
        compute_chunk(q, q % 2)
        copy = nxt

    pltpu.sync_copy(dist_v, dist_hbm.at[pl.ds(wid * BPW, BPW)])
    pltpu.sync_copy(logp_v, logp_hbm.at[pl.ds(wid * BPW, BPW)])


_sc_main = functools.partial(
    pl.kernel,
    out_type=(jax.ShapeDtypeStruct((BATCH,), jnp.float32),
              jax.ShapeDtypeStruct((BATCH,), jnp.float32)),
    mesh=plsc.VectorSubcoreMesh(core_axis_name="c", subcore_axis_name="s"),
    scratch_types=[
        pltpu.VMEM((PW,), jnp.int32),
        pltpu.VMEM((PW,), jnp.int32),
        pltpu.VMEM((CH,), jnp.int32),
        pltpu.VMEM((CH,), jnp.int32),
        pltpu.VMEM((BPW,), jnp.int32),
        pltpu.VMEM((BPW,), jnp.int32),
        pltpu.VMEM((LANES,), jnp.float32),
        pltpu.VMEM((BPW,), jnp.float32),
        pltpu.VMEM((BPW,), jnp.float32),
        pltpu.SemaphoreType.DMA,
        pltpu.SemaphoreType.DMA,
    ],
    compiler_params=pltpu.CompilerParams(needs_layout_passes=False),
)(_sc_main_body)


def _bf16_round(u):
    return (u + jnp.uint32(0x7FFF) + ((u >> 16) & jnp.uint32(1))) \
        & jnp.uint32(0xFFFF0000)


def kernel(edge_adjacency_logits, edge_weight_logits, default_distance,
           reorder, target_paths, from_ix, to_ix):
    tp_flat = target_paths.astype(jnp.int32).reshape(NTOT)
    pidx = _sc_reorder(tp_flat, reorder.astype(jnp.int32))
    wu = lax.bitcast_convert_type(edge_weight_logits, jnp.uint32)
    au = lax.bitcast_convert_type(edge_adjacency_logits, jnp.uint32)
    packed = _bf16_round(wu) | (_bf16_round(au) >> 16)
    packed = lax.bitcast_convert_type(packed, jnp.int32).reshape(E1)
    dflt16 = jnp.broadcast_to(default_distance.reshape(1), (LANES,))
    return _sc_main(tp_flat, pidx, packed,
                    from_ix.astype(jnp.int32),
                    to_ix.astype(jnp.int32),
                    dflt16)


# speedup vs baseline: 1171.9892x; 1.0070x over previous
import functools

import jax
import jax.numpy as jnp
from jax import lax
from jax.experimental import pallas as pl
from jax.experimental.pallas import tpu as pltpu
from jax.experimental.pallas import tpu_sc as plsc

N_EDGES = 3200000
E1 = N_EDGES + 1
BATCH = 16384
PATH_LEN = 32

NC, NS = 2, 16
NW = NC * NS
NTOT = BATCH * PATH_LEN
PW = NTOT // NW
BPW = BATCH // NW
LANES = 16

_LOG1P_C = (3.511021357038846e-06, 0.9997923620654405, -0.49697743071892625,
            0.3145891739884515, -0.1887808235475876, 0.08172564528980446,
            -0.017207799230048133)


def _softplus(x):
    t = jnp.exp(-jnp.abs(x))
    p = jnp.full_like(t, _LOG1P_C[6])
    for c in (_LOG1P_C[5], _LOG1P_C[4], _LOG1P_C[3], _LOG1P_C[2],
              _LOG1P_C[1], _LOG1P_C[0]):
        p = p * t + c
    return jnp.maximum(x, 0.0) + p


def _sc_reorder_body(tp_hbm, reorder_hbm, pidx_hbm, idx_v, pidx_v, sem):
    wid = lax.axis_index("s") * NC + lax.axis_index("c")
    base = wid * PW
    pltpu.sync_copy(tp_hbm.at[pl.ds(base, PW)], idx_v)
    pltpu.async_copy(reorder_hbm.at[idx_v], pidx_v, sem).wait()
    pltpu.sync_copy(pidx_v, pidx_hbm.at[pl.ds(base, PW)])


_sc_reorder = functools.partial(
    pl.kernel,
    out_type=jax.ShapeDtypeStruct((NTOT,), jnp.int32),
    mesh=plsc.VectorSubcoreMesh(core_axis_name="c", subcore_axis_name="s"),
    scratch_types=[
        pltpu.VMEM((PW,), jnp.int32),
        pltpu.VMEM((PW,), jnp.int32),
        pltpu.SemaphoreType.DMA,
    ],
    compiler_params=pltpu.CompilerParams(needs_layout_passes=False),
)(_sc_reorder_body)


CH = 2048
NCH = PW // CH
ROWS_CH = CH // PATH_LEN


def _sc_main_body(tp_hbm, pidx_hbm, pk_hbm, fi_hbm, ti_hbm, dflt_hbm,
                  dist_hbm, logp_hbm,
                  idx_v, pidx_v, pk_bufs, fi_v, ti_v, dflt_v,
                  dist_v, logp_v, sems, sem_st):
    wid = lax.axis_index("s") * NC + lax.axis_index("c")
    base = wid * PW
    pltpu.sync_copy(pidx_hbm.at[pl.ds(base, PW)], pidx_v)
    st = [pltpu.async_copy(tp_hbm.at[pl.ds(base, PW)], idx_v, sem_st),
          pltpu.async_copy(fi_hbm.at[pl.ds(wid * BPW, BPW)], fi_v, sem_st),
          pltpu.async_copy(ti_hbm.at[pl.ds(wid * BPW, BPW)], ti_v, sem_st),
          pltpu.async_copy(dflt_hbm, dflt_v, sem_st)]

    bufs = pk_bufs
    lane = lax.iota(jnp.int32, LANES)
    zero_f = jnp.zeros((LANES,), jnp.float32)
    hi_mask = jnp.full((LANES,), -65536, jnp.int32)

    def gather_chunk(q, buf):
        pidx_q = pidx_v.at[pl.ds(q * CH, CH)]
        return pltpu.async_copy(pk_hbm.at[pidx_q], bufs[buf], sems[buf])

    def compute_chunk(q, buf):
        def grp(g, _):
            bid = (g * LANES + lane) * PATH_LEN

            def inner(l, carry):
                aw, aa = carry
                ids = bid + l
                m = plsc.load_gather(idx_v, [q * CH + ids]) != 0
                pk = plsc.load_gather(bufs[buf], [ids])
                w = plsc.bitcast(pk & hi_mask, jnp.float32)
                a = plsc.bitcast(pk << 16, jnp.float32)
                aw = aw + jnp.where(m, _softplus(w), zero_f)
                aa = aa + jnp.where(m, _softplus(-a), zero_f)
                return aw, aa

            acc_w, acc_a = lax.fori_loop(0, PATH_LEN, inner, (zero_f, zero_f))
            tp0 = plsc.load_gather(idx_v, [q * CH + bid])
            sl = pl.ds(q * ROWS_CH + g * LANES, LANES)
            nf = (tp0 == 0) & (fi_v[sl] != ti_v[sl])
            dist_v[sl] = jnp.where(nf, dflt_v[...], acc_w)
            logp_v[sl] = -acc_a
            return 0

        lax.fori_loop(0, ROWS_CH // LANES, grp, 0)

    copies = [gather_chunk(q, q) for q in range(NCH)]
    for c in st:
        c.wait()
    for q in range(NCH):
        copies[q].wait()
        compute_chunk(q, q)

    pltpu.sync_copy(dist_v, dist_hbm.at[pl.ds(wid * BPW, BPW)])
    pltpu.sync_copy(logp_v, logp_hbm.at[pl.ds(wid * BPW, BPW)])


_sc_main = functools.partial(
    pl.kernel,
    out_type=(jax.ShapeDtypeStruct((BATCH,), jnp.float32),
              jax.ShapeDtypeStruct((BATCH,), jnp.float32)),
    mesh=plsc.VectorSubcoreMesh(core_axis_name="c", subcore_axis_name="s"),
    scratch_types=[
        pltpu.VMEM((PW,), jnp.int32),
        pltpu.VMEM((PW,), jnp.int32),
        [pltpu.VMEM((CH,), jnp.int32)] * NCH,
        pltpu.VMEM((BPW,), jnp.int32),
        pltpu.VMEM((BPW,), jnp.int32),
        pltpu.VMEM((LANES,), jnp.float32),
        pltpu.VMEM((BPW,), jnp.float32),
        pltpu.VMEM((BPW,), jnp.float32),
        [pltpu.SemaphoreType.DMA] * NCH,
        pltpu.SemaphoreType.DMA,
    ],
    compiler_params=pltpu.CompilerParams(needs_layout_passes=False),
)(_sc_main_body)


def _bf16_round(u):
    return (u + jnp.uint32(0x7FFF) + ((u >> 16) & jnp.uint32(1))) \
        & jnp.uint32(0xFFFF0000)


def kernel(edge_adjacency_logits, edge_weight_logits, default_distance,
           reorder, target_paths, from_ix, to_ix):
    tp_flat = target_paths.astype(jnp.int32).reshape(NTOT)
    pidx = _sc_reorder(tp_flat, reorder.astype(jnp.int32))
    wu = lax.bitcast_convert_type(edge_weight_logits, jnp.uint32)
    au = lax.bitcast_convert_type(edge_adjacency_logits, jnp.uint32)
    packed = _bf16_round(wu) | (_bf16_round(au) >> 16)
    packed = lax.bitcast_convert_type(packed, jnp.int32).reshape(E1)
    dflt16 = jnp.broadcast_to(default_distance.reshape(1), (LANES,))
    return _sc_main(tp_flat, pidx, packed,
                    from_ix.astype(jnp.int32),
                    to_ix.astype(jnp.int32),
                    dflt16)


# speedup vs baseline: 1253.0432x; 1.0692x over previous
import functools

import jax
import jax.numpy as jnp
from jax import lax
from jax.experimental import pallas as pl
from jax.experimental.pallas import tpu as pltpu
from jax.experimental.pallas import tpu_sc as plsc

N_EDGES = 3200000
E1 = N_EDGES + 1
BATCH = 16384
PATH_LEN = 32

NC, NS = 2, 16
NW = NC * NS
NTOT = BATCH * PATH_LEN
PW = NTOT // NW
BPW = BATCH // NW
LANES = 16
LBLK = 8
NBLK = PATH_LEN // LBLK

_LOG1P_C = (3.511021357038846e-06, 0.9997923620654405, -0.49697743071892625,
            0.3145891739884515, -0.1887808235475876, 0.08172564528980446,
            -0.017207799230048133)


def _softplus(x):
    t = jnp.exp(-jnp.abs(x))
    p = jnp.full_like(t, _LOG1P_C[6])
    for c in (_LOG1P_C[5], _LOG1P_C[4], _LOG1P_C[3], _LOG1P_C[2],
              _LOG1P_C[1], _LOG1P_C[0]):
        p = p * t + c
    return jnp.maximum(x, 0.0) + p


def _sc_reorder_body(tp_hbm, reorder_hbm, pidx_hbm, idx_v, pidx_v, sem):
    wid = lax.axis_index("s") * NC + lax.axis_index("c")
    base = wid * PW
    pltpu.sync_copy(tp_hbm.at[pl.ds(base, PW)], idx_v)
    pltpu.async_copy(reorder_hbm.at[idx_v], pidx_v, sem).wait()
    pltpu.sync_copy(pidx_v, pidx_hbm.at[pl.ds(base, PW)])


_sc_reorder = functools.partial(
    pl.kernel,
    out_type=jax.ShapeDtypeStruct((NTOT,), jnp.int32),
    mesh=plsc.VectorSubcoreMesh(core_axis_name="c", subcore_axis_name="s"),
    scratch_types=[
        pltpu.VMEM((PW,), jnp.int32),
        pltpu.VMEM((PW,), jnp.int32),
        pltpu.SemaphoreType.DMA,
    ],
    compiler_params=pltpu.CompilerParams(needs_layout_passes=False),
)(_sc_reorder_body)


def _sc_main_body(tp_hbm, pidx_hbm, pk_hbm, fi_hbm, ti_hbm, dflt_hbm,
                  dist_hbm, logp_hbm,
                  idx_bufs, pidx_bufs, pk_bufs, fi_v, ti_v, dflt_v,
                  dist_v, logp_v, sem_g, sem_p, sem_st):
    wid = lax.axis_index("s") * NC + lax.axis_index("c")
    col = pl.ds(wid * BPW, BPW)

    pcopies = [pltpu.async_copy(
        pidx_hbm.at[pl.ds(l * BATCH + wid * BPW, BPW)], pidx_bufs[l], sem_p)
        for l in range(PATH_LEN)]
    st = [pltpu.async_copy(
        tp_hbm.at[pl.ds(l * BATCH + wid * BPW, BPW)], idx_bufs[l], sem_st)
        for l in range(PATH_LEN)]
    st += [pltpu.async_copy(fi_hbm.at[col], fi_v, sem_st),
           pltpu.async_copy(ti_hbm.at[col], ti_v, sem_st),
           pltpu.async_copy(dflt_hbm, dflt_v, sem_st)]

    gcopies = []
    for l in range(PATH_LEN):
        pcopies[l].wait()
        gcopies.append(
            pltpu.async_copy(pk_hbm.at[pidx_bufs[l]], pk_bufs[l], sem_g))
    for c in st:
        c.wait()

    lane = lax.iota(jnp.int32, LANES)
    zero_f = jnp.zeros((LANES,), jnp.float32)
    hi_mask = jnp.full((LANES,), -65536, jnp.int32)

    def block(qb):
        def grp(g, _):
            bid = g * LANES + lane
            acc_w, acc_a = zero_f, zero_f
            for dl in range(LBLK):
                l = qb * LBLK + dl
                m = plsc.load_gather(idx_bufs[l], [bid]) != 0
                pk = plsc.load_gather(pk_bufs[l], [bid])
                w = plsc.bitcast(pk & hi_mask, jnp.float32)
                a = plsc.bitcast(pk << 16, jnp.float32)
                acc_w = acc_w + jnp.where(m, _softplus(w), zero_f)
                acc_a = acc_a + jnp.where(m, _softplus(-a), zero_f)
            sl = pl.ds(g * LANES, LANES)
            if qb == 0:
                dist_v[sl] = acc_w
                logp_v[sl] = acc_a
            else:
                dist_v[sl] = dist_v[sl] + acc_w
                logp_v[sl] = logp_v[sl] + acc_a
            return 0

        lax.fori_loop(0, BPW // LANES, grp, 0)

    for qb in range(NBLK):
        for dl in range(LBLK):
            gcopies[qb * LBLK + dl].wait()
        block(qb)

    def fin(g, _):
        sl = pl.ds(g * LANES, LANES)
        nf = (idx_bufs[0][sl] == 0) & (fi_v[sl] != ti_v[sl])
        dist_v[sl] = jnp.where(nf, dflt_v[...], dist_v[sl])
        logp_v[sl] = -logp_v[sl]
        return 0

    lax.fori_loop(0, BPW // LANES, fin, 0)
    pltpu.sync_copy(dist_v, dist_hbm.at[col])
    pltpu.sync_copy(logp_v, logp_hbm.at[col])


_sc_main = functools.partial(
    pl.kernel,
    out_type=(jax.ShapeDtypeStruct((BATCH,), jnp.float32),
              jax.ShapeDtypeStruct((BATCH,), jnp.float32)),
    mesh=plsc.VectorSubcoreMesh(core_axis_name="c", subcore_axis_name="s"),
    scratch_types=[
        [pltpu.VMEM((BPW,), jnp.int32)] * PATH_LEN,
        [pltpu.VMEM((BPW,), jnp.int32)] * PATH_LEN,
        [pltpu.VMEM((BPW,), jnp.int32)] * PATH_LEN,
        pltpu.VMEM((BPW,), jnp.int32),
        pltpu.VMEM((BPW,), jnp.int32),
        pltpu.VMEM((LANES,), jnp.float32),
        pltpu.VMEM((BPW,), jnp.float32),
        pltpu.VMEM((BPW,), jnp.float32),
        pltpu.SemaphoreType.DMA,
        pltpu.SemaphoreType.DMA,
        pltpu.SemaphoreType.DMA,
    ],
    compiler_params=pltpu.CompilerParams(needs_layout_passes=False),
)(_sc_main_body)


def _bf16_round(u):
    return (u + jnp.uint32(0x7FFF) + ((u >> 16) & jnp.uint32(1))) \
        & jnp.uint32(0xFFFF0000)


def kernel(edge_adjacency_logits, edge_weight_logits, default_distance,
           reorder, target_paths, from_ix, to_ix):
    tp_lm = target_paths.astype(jnp.int32).T.reshape(NTOT)
    pidx = _sc_reorder(tp_lm, reorder.astype(jnp.int32))
    wu = lax.bitcast_convert_type(edge_weight_logits, jnp.uint32)
    au = lax.bitcast_convert_type(edge_adjacency_logits, jnp.uint32)
    packed = _bf16_round(wu) | (_bf16_round(au) >> 16)
    packed = lax.bitcast_convert_type(packed, jnp.int32).reshape(E1)
    dflt16 = jnp.broadcast_to(default_distance.reshape(1), (LANES,))
    return _sc_main(tp_lm, pidx, packed,
                    from_ix.astype(jnp.int32),
                    to_ix.astype(jnp.int32),
                    dflt16)
